# Initial kernel scaffold; baseline (speedup 1.0000x reference)
import jax
import jax.numpy as jnp
from jax.experimental import pallas as pl


def kernel(x, table):
    raise NotImplementedError("write your pallas kernel here")


# speedup vs baseline: 4.5978x; 4.5978x over previous
import functools
import math

import jax
import jax.numpy as jnp
from jax import lax
from jax.experimental import pallas as pl
from jax.experimental.pallas import tpu as pltpu
from jax.experimental.pallas import tpu_sc as plsc

D_MODEL = 768
SCALE = math.sqrt(768.0)

NC = 2
NS = 16
L = 16
NW = NC * NS

B = 4 * 4096
B_PER_W = B // NW
CH = 64
NCHUNK = B_PER_W // CH
NB = 2

_mesh = plsc.VectorSubcoreMesh(core_axis_name="c", subcore_axis_name="s")


@functools.partial(
    pl.kernel,
    mesh=_mesh,
    out_type=jax.ShapeDtypeStruct((B, D_MODEL), jnp.float32),
    scratch_types=[
        pltpu.VMEM((NCHUNK, CH), jnp.int32),
        pltpu.VMEM((NB, CH, D_MODEL), jnp.float32),
        pltpu.SemaphoreType.DMA,
        pltpu.SemaphoreType.DMA,
        pltpu.SemaphoreType.DMA,
        pltpu.SemaphoreType.DMA,
    ],
)
def _embed_sc(x_hbm, table_hbm, out_hbm, idx_v, rows_v, g0, g1, s0, s1):
    wid = lax.axis_index("s") * NC + lax.axis_index("c")
    base = wid * B_PER_W
    gsems = (g0, g1)
    ssems = (s0, s1)

    pltpu.sync_copy(x_hbm.at[wid], idx_v)

    def start_gather(c, b):
        cp = pltpu.make_async_copy(
            table_hbm.at[idx_v.at[c]], rows_v.at[b], gsems[b])
        cp.start()
        return cp

    def start_store(c, b):
        cp = pltpu.make_async_copy(
            rows_v.at[b], out_hbm.at[pl.ds(base + c * CH, CH)], ssems[b])
        cp.start()
        return cp

    def scale_buf(b):
        def row_body(r, carry):
            for dd in range(D_MODEL // L):
                sl = pl.ds(dd * L, L)
                rows_v[b, r, sl] = rows_v[b, r, sl] * SCALE
            return carry
        lax.fori_loop(0, CH, row_body, 0)

    gathers = [None] * NB
    stores = [None] * NB
    gathers[0] = start_gather(0, 0)
    for c in range(NCHUNK):
        b = c % NB
        nb_ = (c + 1) % NB
        if c + 1 < NCHUNK:
            if stores[nb_] is not None:
                stores[nb_].wait()
                stores[nb_] = None
            gathers[nb_] = start_gather(c + 1, nb_)
        gathers[b].wait()
        scale_buf(b)
        stores[b] = start_store(c, b)
    for b in range(NB):
        if stores[b] is not None:
            stores[b].wait()


def kernel(x, table):
    x_blocked = x.reshape(NW, NCHUNK, CH)
    out = _embed_sc(x_blocked, table)
    return out.reshape(x.shape[0], x.shape[1], D_MODEL)
